# baseline (speedup 1.0000x reference)
import functools

import jax
import jax.numpy as jnp
from jax import lax
from jax.experimental import pallas as pl
from jax.experimental.pallas import tpu as pltpu
from jax.experimental.pallas import tpu_sc as plsc

V = 1000
D = 1000
B = 16384

NC = 2
NS = 16
NW = NC * NS
BPW = B // NW
CH = 32
NCH = BPW // CH

STG = 64


def _sigmoid_body(x_ref, o_ref):
    o_ref[...] = jax.nn.sigmoid(x_ref[...])


def _sigmoid_table(logits):
    return pl.pallas_call(
        _sigmoid_body,
        out_shape=jax.ShapeDtypeStruct((V, D), jnp.float32),
    )(logits)


_mesh = plsc.VectorSubcoreMesh(core_axis_name="c", subcore_axis_name="s")


@functools.partial(
    pl.kernel,
    mesh=_mesh,
    out_type=jax.ShapeDtypeStruct((B, D), jnp.float32),
    compiler_params=pltpu.CompilerParams(use_tc_tiling_on_sc=False),
    scratch_types=[
        pltpu.VMEM((NCH, CH), jnp.int32),
        pltpu.VMEM((CH, D), jnp.float32),
        pltpu.VMEM((CH, D), jnp.float32),
        pltpu.VMEM_SHARED((V, D), jnp.float32),
        pltpu.SemaphoreType.DMA,
        pltpu.SemaphoreType.DMA,
    ],
)
def _lookup(r_hbm, idx_hbm, out_hbm, idx_v, buf0, buf1, table_s, sem0, sem1):
    c = lax.axis_index("c")
    s = lax.axis_index("s")
    wid = c * NS + s

    start = jnp.minimum(s * STG, ((V - STG) // 8) * 8)
    pltpu.sync_copy(r_hbm.at[pl.ds(start, STG)], table_s.at[pl.ds(start, STG)])
    plsc.subcore_barrier()

    pltpu.sync_copy(idx_hbm.at[wid], idx_v)

    base = wid * BPW
    bufs = (buf0, buf1)
    sems = (sem0, sem1)
    copies = [None] * NCH
    copies[0] = pltpu.async_copy(table_s.at[idx_v.at[0]], bufs[0], sems[0])
    for cc in range(NCH):
        if cc + 1 < NCH:
            copies[cc + 1] = pltpu.async_copy(
                table_s.at[idx_v.at[cc + 1]], bufs[(cc + 1) % 2], sems[(cc + 1) % 2]
            )
        copies[cc].wait()
        pltpu.sync_copy(bufs[cc % 2], out_hbm.at[pl.ds(base + cc * CH, CH)])


def kernel(idx, morphisms_logits):
    r = _sigmoid_table(morphisms_logits)
    idx3 = idx.astype(jnp.int32).reshape(NW, NCH, CH)
    return _lookup(r, idx3)


# speedup vs baseline: 1.8687x; 1.8687x over previous
import functools

import jax
import jax.numpy as jnp
from jax import lax
from jax.experimental import pallas as pl

V = 1000
B = 16384
BT = 256
NJ = B // BT


def _prep_body(x_ref, hi_ref, lo_ref):
    r = jax.nn.sigmoid(x_ref[...])
    hi = r.astype(jnp.bfloat16)
    lo = (r - hi.astype(jnp.float32)).astype(jnp.bfloat16)
    hi_ref[...] = hi
    lo_ref[...] = lo


def _prep(logits):
    return pl.pallas_call(
        _prep_body,
        out_shape=(
            jax.ShapeDtypeStruct((V, V), jnp.bfloat16),
            jax.ShapeDtypeStruct((V, V), jnp.bfloat16),
        ),
    )(logits)


def _emb_body(idx_ref, hi_ref, lo_ref, o_ref):
    ids = idx_ref[0, 0, :]
    iot = lax.broadcasted_iota(jnp.int32, (V, BT), 0)
    oh = (iot == ids[None, :]).astype(jnp.bfloat16)
    dn = (((0,), (0,)), ((), ()))
    acc = lax.dot_general(hi_ref[...], oh, dn, preferred_element_type=jnp.float32)
    acc = acc + lax.dot_general(lo_ref[...], oh, dn, preferred_element_type=jnp.float32)
    o_ref[...] = acc


def _emb(idx3, hi, lo):
    return pl.pallas_call(
        _emb_body,
        grid=(NJ,),
        in_specs=[
            pl.BlockSpec((1, 1, BT), lambda j: (j, 0, 0)),
            pl.BlockSpec((V, V), lambda j: (0, 0)),
            pl.BlockSpec((V, V), lambda j: (0, 0)),
        ],
        out_specs=pl.BlockSpec((V, BT), lambda j: (0, j)),
        out_shape=jax.ShapeDtypeStruct((V, B), jnp.float32),
    )(idx3, hi, lo)


def kernel(idx, morphisms_logits):
    hi, lo = _prep(morphisms_logits)
    idx3 = idx.astype(jnp.int32).reshape(NJ, 1, BT)
    ot = _emb(idx3, hi, lo)
    return ot.T
